# baseline (speedup 1.0000x reference)
import jax
import jax.numpy as jnp
from jax.experimental import pallas as pl
from jax.experimental.pallas import tpu as pltpu

_SIGMA = 1000000.0
_DT = 0.0005
_XMIN, _XMAX = 32, 48
_YMIN, _YMAX = 56, 72
_ZMIN, _ZMAX = 56, 72
_N = 128
_INV = 1.0 / (1.0 + _DT * _SIGMA)
_NZ = _ZMAX - _ZMIN
_NY = _YMAX - _YMIN


def _body(u_ref, v_ref, w_ref, ou_ref, ov_ref, ow_ref,
          su, sv, sw, sems):
    ins = (u_ref, v_ref, w_ref)
    outs = (ou_ref, ov_ref, ow_ref)
    scratch = (su, sv, sw)

    copies = []
    k = 0
    for f in range(3):
        src, dst = ins[f], outs[f]
        for sl in (
            (pl.ds(0, _ZMIN),),
            (pl.ds(_ZMAX, _N - _ZMAX),),
            (pl.ds(_ZMIN, _NZ), pl.ds(0, _YMIN)),
            (pl.ds(_ZMIN, _NZ), pl.ds(_YMAX, _N - _YMAX)),
        ):
            copies.append(pltpu.make_async_copy(
                src.at[sl], dst.at[sl], sems.at[k]))
            k += 1
        copies.append(pltpu.make_async_copy(
            src.at[pl.ds(_ZMIN, _NZ), pl.ds(_YMIN, _NY)],
            scratch[f], sems.at[k]))
        k += 1
    for c in copies:
        c.start()

    xi = jax.lax.broadcasted_iota(jnp.int32, (_NZ, _NY, _N), 2)
    scale = jnp.where((xi >= _XMIN) & (xi < _XMAX),
                      jnp.float32(_INV), jnp.float32(1.0))
    wb = []
    for f in range(3):
        copies[4 * f + 4 + f].wait()
        scratch[f][...] = scratch[f][...] * scale
        c = pltpu.make_async_copy(
            scratch[f],
            outs[f].at[pl.ds(_ZMIN, _NZ), pl.ds(_YMIN, _NY)],
            sems.at[k])
        k += 1
        c.start()
        wb.append(c)
    for f in range(3):
        for j in range(4):
            copies[5 * f + j].wait()
    for c in wb:
        c.wait()


def kernel(values_u, values_v, values_w):
    u3 = values_u.reshape(_N, _N, _N)
    v3 = values_v.reshape(_N, _N, _N)
    w3 = values_w.reshape(_N, _N, _N)

    any_spec = pl.BlockSpec(memory_space=pl.ANY)
    out = pl.pallas_call(
        _body,
        in_specs=[any_spec] * 3,
        out_specs=[any_spec] * 3,
        out_shape=[jax.ShapeDtypeStruct((_N, _N, _N), jnp.float32)] * 3,
        scratch_shapes=[pltpu.VMEM((_NZ, _NY, _N), jnp.float32)] * 3
        + [pltpu.SemaphoreType.DMA((18,))],
    )(u3, v3, w3)
    shp = values_u.shape
    return tuple(o.reshape(shp) for o in out)


# speedup vs baseline: 46.1384x; 46.1384x over previous
import jax
import jax.numpy as jnp
from jax.experimental import pallas as pl
from jax.experimental.pallas import tpu as pltpu

_SIGMA = 1000000.0
_DT = 0.0005
_XMIN, _XMAX = 32, 48
_YMIN, _YMAX = 56, 72
_ZMIN, _ZMAX = 56, 72
_N = 128
_BZ = 64
_INV = 1.0 / (1.0 + _DT * _SIGMA)


def _body(u_ref, v_ref, w_ref, ou_ref, ov_ref, ow_ref):
    i = pl.program_id(0)
    z0 = i * _BZ
    touches = (z0 < _ZMAX) & (z0 + _BZ > _ZMIN)

    @pl.when(jnp.logical_not(touches))
    def _copy():
        ou_ref[...] = u_ref[...]
        ov_ref[...] = v_ref[...]
        ow_ref[...] = w_ref[...]

    @pl.when(touches)
    def _masked():
        zi = jax.lax.broadcasted_iota(jnp.int32, (_BZ, _N, _N), 0) + z0
        yi = jax.lax.broadcasted_iota(jnp.int32, (_BZ, _N, _N), 1)
        xi = jax.lax.broadcasted_iota(jnp.int32, (_BZ, _N, _N), 2)
        inside = (
            (zi >= _ZMIN) & (zi < _ZMAX)
            & (yi >= _YMIN) & (yi < _YMAX)
            & (xi >= _XMIN) & (xi < _XMAX)
        )
        scale = jnp.where(inside, jnp.float32(_INV), jnp.float32(1.0))
        ou_ref[...] = u_ref[...] * scale
        ov_ref[...] = v_ref[...] * scale
        ow_ref[...] = w_ref[...] * scale


def kernel(values_u, values_v, values_w):
    u3 = values_u.reshape(_N, _N, _N)
    v3 = values_v.reshape(_N, _N, _N)
    w3 = values_w.reshape(_N, _N, _N)

    spec = pl.BlockSpec((_BZ, _N, _N), lambda i: (i, 0, 0))
    out = pl.pallas_call(
        _body,
        grid=(_N // _BZ,),
        in_specs=[spec, spec, spec],
        out_specs=[spec, spec, spec],
        out_shape=[jax.ShapeDtypeStruct((_N, _N, _N), jnp.float32)] * 3,
        compiler_params=pltpu.CompilerParams(
            dimension_semantics=("arbitrary",),
        ),
    )(u3, v3, w3)
    shp = values_u.shape
    return tuple(o.reshape(shp) for o in out)
